# Initial kernel scaffold; baseline (speedup 1.0000x reference)
import jax
import jax.numpy as jnp
from jax.experimental import pallas as pl


def kernel(inputs, emb_table, W, U, b, Wd, bd):
    raise NotImplementedError("write your pallas kernel here")


# speedup vs baseline: 13.4077x; 13.4077x over previous
import functools

import jax
import jax.numpy as jnp
from jax import lax
from jax.experimental import pallas as pl
from jax.experimental.pallas import tpu as pltpu
from jax.experimental.pallas import tpu_sc as plsc

B, T, V, D, H = 4096, 200, 1000000, 32, 128

_NC, _NS = 2, 16
_NW = _NC * _NS
_ROWS = B * T
_PER_W = _ROWS // _NW
_K = 8
_G = _K * 128
_ITERS = _PER_W // _G
_IDXR = _PER_W // 128


def _sc_gather(table, idx2d):
    mesh = plsc.VectorSubcoreMesh(core_axis_name="c", subcore_axis_name="s")

    @functools.partial(
        pl.kernel,
        mesh=mesh,
        compiler_params=pltpu.CompilerParams(use_tc_tiling_on_sc=False),
        out_type=jax.ShapeDtypeStruct((_ROWS, D), jnp.float32),
        scratch_types=[
            pltpu.VMEM((_IDXR, 128), jnp.int32),
            pltpu.VMEM((_G, D), jnp.float32),
            pltpu.SemaphoreType.DMA,
        ],
    )
    def k(table_hbm, idx_hbm, out_hbm, idx_v, rows_v, sem):
        wid = lax.axis_index("s") * _NC + lax.axis_index("c")
        row0 = wid * _PER_W
        irow0 = wid * _IDXR

        pltpu.sync_copy(idx_hbm.at[pl.ds(irow0, _IDXR)], idx_v)

        def body(i, carry):
            copies = []
            for j in range(_K):
                copies.append(
                    pltpu.async_copy(
                        table_hbm.at[idx_v.at[i * _K + j]],
                        rows_v.at[pl.ds(j * 128, 128)],
                        sem,
                    )
                )
            for cpy in copies:
                cpy.wait()
            pltpu.sync_copy(rows_v, out_hbm.at[pl.ds(row0 + i * _G, _G)])
            return carry

        lax.fori_loop(0, _ITERS, body, 0)

    return k(table, idx2d)


_BT = 1024
_NB = B // _BT


def _lstm_body(xs_ref, W_ref, U_ref, b_ref, Wd_ref, bd_ref, out_ref,
               h_scr, c_scr):
    t = pl.program_id(1)

    @pl.when(t == 0)
    def _():
        h_scr[...] = jnp.zeros_like(h_scr)
        c_scr[...] = jnp.zeros_like(c_scr)

    xt = xs_ref[0]
    h = h_scr[...]
    c = c_scr[...]
    z = (jnp.dot(xt, W_ref[...], preferred_element_type=jnp.float32)
         + jnp.dot(h, U_ref[...], preferred_element_type=jnp.float32)
         + b_ref[...])
    i_g = jax.nn.sigmoid(z[:, 0 * H:1 * H])
    f_g = jax.nn.sigmoid(z[:, 1 * H:2 * H])
    g_g = jnp.tanh(z[:, 2 * H:3 * H])
    o_g = jax.nn.sigmoid(z[:, 3 * H:4 * H])
    c_new = f_g * c + i_g * g_g
    h_new = o_g * jnp.tanh(c_new)
    h_scr[...] = h_new
    c_scr[...] = c_new

    @pl.when(t == T - 1)
    def _():
        out_ref[...] = jax.nn.sigmoid(
            jnp.dot(h_new, Wd_ref[...], preferred_element_type=jnp.float32)
            + bd_ref[...])


def _lstm(xs, W, U, b2, Wd, bd2):
    return pl.pallas_call(
        _lstm_body,
        grid=(_NB, T),
        in_specs=[
            pl.BlockSpec((1, _BT, D), lambda i, t: (t, i, 0)),
            pl.BlockSpec((D, 4 * H), lambda i, t: (0, 0)),
            pl.BlockSpec((H, 4 * H), lambda i, t: (0, 0)),
            pl.BlockSpec((1, 4 * H), lambda i, t: (0, 0)),
            pl.BlockSpec((H, 1), lambda i, t: (0, 0)),
            pl.BlockSpec((1, 1), lambda i, t: (0, 0)),
        ],
        out_specs=pl.BlockSpec((_BT, 1), lambda i, t: (i, 0)),
        out_shape=jax.ShapeDtypeStruct((B, 1), jnp.float32),
        scratch_shapes=[
            pltpu.VMEM((_BT, H), jnp.float32),
            pltpu.VMEM((_BT, H), jnp.float32),
        ],
    )(xs, W, U, b2, Wd, bd2)


def kernel(inputs, emb_table, W, U, b, Wd, bd):
    idx = jnp.asarray(inputs, jnp.int32).T.reshape(_ROWS // 128, 128)
    xs_flat = _sc_gather(emb_table, idx)
    xs = xs_flat.reshape(T, B, D)
    return _lstm(xs, W, U, b.reshape(1, 4 * H), Wd, bd.reshape(1, 1))
